# baseline (speedup 1.0000x reference)
import functools

import jax
import jax.numpy as jnp
from jax import lax
from jax.experimental import pallas as pl
from jax.experimental.pallas import tpu as pltpu
from jax.experimental.pallas import tpu_sc as plsc

_N_THETA = 16
_N_BINS = 128
_LANES = 16
_OB_PITCH = _N_BINS + 1


def kernel(descriptor, r_array_q, theta_array_q, sum_points):
    B, N, NB = descriptor.shape
    R = B * N
    NW = 32
    G = _LANES
    rows_per_w = R // NW
    groups_per_w = rows_per_w // G
    n_iters = groups_per_w // 2

    r2 = r_array_q.reshape(R, N)
    t2 = theta_array_q.reshape(R, N)
    d2 = descriptor.reshape(R, NB)

    inv = 1.0 / sum_points.astype(jnp.float32)
    inv_w = jnp.repeat(inv, NW // B)
    inv_splat = jnp.broadcast_to(inv_w[:, None], (NW, _LANES))

    mesh = plsc.VectorSubcoreMesh(core_axis_name="c", subcore_axis_name="s")

    @functools.partial(
        pl.kernel,
        out_type=jax.ShapeDtypeStruct((R, NB), jnp.float32),
        mesh=mesh,
        scratch_types=[
            pltpu.VMEM((G, N), jnp.int32),
            pltpu.VMEM((G, N), jnp.int32),
            pltpu.VMEM((G, N), jnp.int32),
            pltpu.VMEM((G, N), jnp.int32),
            pltpu.VMEM((NB, G), jnp.float32),
            pltpu.VMEM((G, _OB_PITCH), jnp.float32),
            pltpu.VMEM((G, _OB_PITCH), jnp.float32),
            pltpu.VMEM((_LANES,), jnp.float32),
            pltpu.SemaphoreType.DMA,
            pltpu.SemaphoreType.DMA,
            pltpu.SemaphoreType.DMA,
            pltpu.SemaphoreType.DMA,
            pltpu.SemaphoreType.DMA,
            pltpu.SemaphoreType.DMA,
        ],
        compiler_params=pltpu.CompilerParams(needs_layout_passes=False),
    )
    def run(d_hbm, r_hbm, t_hbm, inv_hbm, out_hbm,
            rb0, rb1, tb0, tb1, acct, ob0, ob1, invv,
            isem0, isem1, dsem0, dsem1, osem0, osem1):
        wid = lax.axis_index("s") * 2 + lax.axis_index("c")
        pltpu.sync_copy(inv_hbm.at[wid], invv)
        ival = invv[...]
        rb = (rb0, rb1)
        tb = (tb0, tb1)
        ob = (ob0, ob1)
        isem = (isem0, isem1)
        dsem = (dsem0, dsem1)
        osem = (osem0, osem1)
        w_row0 = wid * rows_per_w
        iota = lax.iota(jnp.int32, _LANES)
        zero16 = jnp.zeros((_LANES,), jnp.float32)

        def fire_in(g, buf):
            row = w_row0 + g * G
            pltpu.async_copy(r_hbm.at[pl.ds(row, G)], rb[buf], isem[buf])
            pltpu.async_copy(t_hbm.at[pl.ds(row, G)], tb[buf], isem[buf])

        def fire_desc(g, buf):
            row = w_row0 + g * G
            pltpu.async_copy(d_hbm.at[pl.ds(row, G)],
                             ob[buf].at[:, pl.ds(0, NB)], dsem[buf])

        def wait_in(g, buf):
            row = w_row0 + g * G
            pltpu.make_async_copy(r_hbm.at[pl.ds(row, G)], rb[buf], isem[buf]).wait()
            pltpu.make_async_copy(t_hbm.at[pl.ds(row, G)], tb[buf], isem[buf]).wait()
            pltpu.make_async_copy(d_hbm.at[pl.ds(row, G)],
                                  ob[buf].at[:, pl.ds(0, NB)], dsem[buf]).wait()

        def fire_out(g, buf):
            row = w_row0 + g * G
            pltpu.async_copy(ob[buf].at[:, pl.ds(0, NB)],
                             out_hbm.at[pl.ds(row, G)], osem[buf])

        def wait_out(buf):
            pltpu.make_async_copy(d_hbm.at[pl.ds(0, G)],
                                  ob[buf].at[:, pl.ds(0, NB)], osem[buf]).wait()

        def compute(buf):
            rbr, tbr, obr = rb[buf], tb[buf], ob[buf]

            @plsc.parallel_loop(0, N, 8)
            def col_body(j):
                jvec = jnp.full((_LANES,), j, jnp.int32) + iota
                for u in range(8):
                    acol = (jvec + u) & (N - 1)
                    rv = plsc.load_gather(rbr, [iota, acol])
                    tv = plsc.load_gather(tbr, [iota, acol])
                    bins = (rv << 4) + tv
                    plsc.addupdate_scatter(acct, [bins, iota], ival)

            @plsc.parallel_loop(0, NB, 1)
            def trans_body(s):
                v = acct[s, :]
                plsc.addupdate_scatter(
                    obr, [iota, jnp.full((_LANES,), s, jnp.int32)], v)
                acct[s, :] = zero16

        @plsc.parallel_loop(0, NB, 1)
        def zero_body(s):
            acct[s, :] = zero16

        fire_in(0, 0)
        fire_desc(0, 0)

        def step(k, carry):
            g0 = 2 * k
            g1 = g0 + 1
            fire_in(g1, 1)
            wait_in(g0, 0)
            compute(0)

            @pl.when(k >= 1)
            def _():
                wait_out(1)
            fire_desc(g1, 1)
            fire_out(g0, 0)

            @pl.when(k < n_iters - 1)
            def _():
                fire_in(g0 + 2, 0)
            wait_in(g1, 1)
            compute(1)

            @pl.when(k < n_iters - 1)
            def _():
                wait_out(0)
                fire_desc(g0 + 2, 0)
            fire_out(g1, 1)
            return carry

        lax.fori_loop(0, n_iters, step, 0)
        wait_out(0)
        wait_out(1)

    return run(d2, r2, t2, inv_splat).reshape(B, N, NB)


# speedup vs baseline: 1.1199x; 1.1199x over previous
import functools

import jax
import jax.numpy as jnp
from jax import lax
from jax.experimental import pallas as pl
from jax.experimental.pallas import tpu as pltpu
from jax.experimental.pallas import tpu_sc as plsc

_N_THETA = 16
_N_BINS = 128
_LANES = 16


def kernel(descriptor, r_array_q, theta_array_q, sum_points):
    B, N, NB = descriptor.shape
    R = B * N
    NW = 32
    G = _LANES
    rows_per_w = R // NW
    groups_per_w = rows_per_w // G
    n_iters = groups_per_w // 2
    m_main = N - G

    r2 = r_array_q.reshape(R, N)
    t2 = theta_array_q.reshape(R, N)
    d2 = descriptor.reshape(R, NB)

    inv = 1.0 / sum_points.astype(jnp.float32)
    inv_w = jnp.repeat(inv, NW // B)
    inv_splat = jnp.broadcast_to(inv_w[:, None], (NW, _LANES))

    mesh = plsc.VectorSubcoreMesh(core_axis_name="c", subcore_axis_name="s")

    @functools.partial(
        pl.kernel,
        out_type=jax.ShapeDtypeStruct((R, NB), jnp.float32),
        mesh=mesh,
        scratch_types=[
            pltpu.VMEM((G * N,), jnp.int32),
            pltpu.VMEM((G * N,), jnp.int32),
            pltpu.VMEM((G * N,), jnp.int32),
            pltpu.VMEM((G * N,), jnp.int32),
            pltpu.VMEM((NB * G,), jnp.float32),
            pltpu.VMEM((G, NB), jnp.float32),
            pltpu.VMEM((G, NB), jnp.float32),
            pltpu.VMEM((_LANES,), jnp.float32),
            pltpu.SemaphoreType.DMA,
            pltpu.SemaphoreType.DMA,
            pltpu.SemaphoreType.DMA,
            pltpu.SemaphoreType.DMA,
            pltpu.SemaphoreType.DMA,
            pltpu.SemaphoreType.DMA,
        ],
        compiler_params=pltpu.CompilerParams(needs_layout_passes=False),
    )
    def run(d_hbm, r_hbm, t_hbm, inv_hbm, out_hbm,
            rb0, rb1, tb0, tb1, acct, ob0, ob1, invv,
            isem0, isem1, dsem0, dsem1, osem0, osem1):
        wid = lax.axis_index("s") * 2 + lax.axis_index("c")
        pltpu.sync_copy(inv_hbm.at[wid], invv)
        ival = invv[...]
        rb = (rb0, rb1)
        tb = (tb0, tb1)
        ob = (ob0, ob1)
        isem = (isem0, isem1)
        dsem = (dsem0, dsem1)
        osem = (osem0, osem1)
        w_row0 = wid * rows_per_w
        iota = lax.iota(jnp.int32, _LANES)
        base_diag = iota * (N + 1)
        zero16 = jnp.zeros((_LANES,), jnp.float32)

        def fire_in(g, buf):
            row = w_row0 + g * G
            for l in range(G):
                pltpu.async_copy(r_hbm.at[row + l],
                                 rb[buf].at[pl.ds(l * N, N)], isem[buf])
                pltpu.async_copy(t_hbm.at[row + l],
                                 tb[buf].at[pl.ds(l * N, N)], isem[buf])

        def fire_desc(g, buf):
            row = w_row0 + g * G
            pltpu.async_copy(d_hbm.at[pl.ds(row, G)], ob[buf], dsem[buf])

        def wait_in(g, buf):
            row = w_row0 + g * G
            for l in range(G):
                pltpu.make_async_copy(r_hbm.at[row + l],
                                      rb[buf].at[pl.ds(l * N, N)], isem[buf]).wait()
                pltpu.make_async_copy(t_hbm.at[row + l],
                                      tb[buf].at[pl.ds(l * N, N)], isem[buf]).wait()
            pltpu.make_async_copy(d_hbm.at[pl.ds(row, G)], ob[buf], dsem[buf]).wait()

        def fire_out(g, buf):
            row = w_row0 + g * G
            pltpu.async_copy(ob[buf], out_hbm.at[pl.ds(row, G)], osem[buf])

        def wait_out(buf):
            pltpu.make_async_copy(d_hbm.at[pl.ds(0, G)], ob[buf], osem[buf]).wait()

        def scat(rbr, tbr, aflat):
            rv = plsc.load_gather(rbr, [aflat])
            tv = plsc.load_gather(tbr, [aflat])
            sidx = (rv << 8) + (tv << 4) + iota
            plsc.addupdate_scatter(acct, [sidx], ival)

        def compute(buf):
            rbr, tbr, obr = rb[buf], tb[buf], ob[buf]

            @plsc.parallel_loop(0, m_main, 8)
            def col_body(m):
                mvec = base_diag + m
                for u in range(8):
                    scat(rbr, tbr, mvec + u)

            @plsc.parallel_loop(m_main, N, 1)
            def tail_body(m):
                aflat = base_diag + m - jnp.where(iota + m >= N, N, 0)
                scat(rbr, tbr, aflat)

            @plsc.parallel_loop(0, NB, 1)
            def trans_body(s):
                t16 = s & (NB - G)
                d = s & (G - 1)
                bvec = ((iota + d) & (G - 1)) + t16
                v = plsc.load_gather(acct, [(bvec << 4) + iota])
                plsc.addupdate_scatter(obr, [iota, bvec], v)

            @plsc.parallel_loop(0, NB, 1)
            def zero_body(s):
                off = pl.multiple_of(s * G, G)
                acct[pl.ds(off, G)] = zero16

        @plsc.parallel_loop(0, NB, 1)
        def zero_init(s):
            off = pl.multiple_of(s * G, G)
            acct[pl.ds(off, G)] = zero16

        fire_in(0, 0)
        fire_desc(0, 0)

        def step(k, carry):
            g0 = 2 * k
            g1 = g0 + 1
            fire_in(g1, 1)
            wait_in(g0, 0)
            compute(0)

            @pl.when(k >= 1)
            def _():
                wait_out(1)
            fire_desc(g1, 1)
            fire_out(g0, 0)

            @pl.when(k < n_iters - 1)
            def _():
                fire_in(g0 + 2, 0)
            wait_in(g1, 1)
            compute(1)

            @pl.when(k < n_iters - 1)
            def _():
                wait_out(0)
                fire_desc(g0 + 2, 0)
            fire_out(g1, 1)
            return carry

        lax.fori_loop(0, n_iters, step, 0)
        wait_out(0)
        wait_out(1)

    return run(d2, r2, t2, inv_splat).reshape(B, N, NB)


# speedup vs baseline: 1.1216x; 1.0015x over previous
import functools

import jax
import jax.numpy as jnp
from jax import lax
from jax.experimental import pallas as pl
from jax.experimental.pallas import tpu as pltpu
from jax.experimental.pallas import tpu_sc as plsc

_N_THETA = 16
_N_BINS = 128
_LANES = 16


def kernel(descriptor, r_array_q, theta_array_q, sum_points):
    B, N, NB = descriptor.shape
    R = B * N
    NW = 32
    G = _LANES
    rows_per_w = R // NW
    groups_per_w = rows_per_w // G
    n_iters = groups_per_w // 2
    m_main = N - G

    r2 = r_array_q.reshape(R, N)
    t2 = theta_array_q.reshape(R, N)
    d2 = descriptor.reshape(R, NB)

    inv = 1.0 / sum_points.astype(jnp.float32)
    inv_w = jnp.repeat(inv, NW // B)
    inv_splat = jnp.broadcast_to(inv_w[:, None], (NW, _LANES))

    mesh = plsc.VectorSubcoreMesh(core_axis_name="c", subcore_axis_name="s")

    @functools.partial(
        pl.kernel,
        out_type=jax.ShapeDtypeStruct((R, NB), jnp.float32),
        mesh=mesh,
        scratch_types=[
            pltpu.VMEM((G * N,), jnp.int32),
            pltpu.VMEM((G * N,), jnp.int32),
            pltpu.VMEM((G * N,), jnp.int32),
            pltpu.VMEM((G * N,), jnp.int32),
            pltpu.VMEM((NB * G,), jnp.float32),
            pltpu.VMEM((G, NB), jnp.float32),
            pltpu.VMEM((G, NB), jnp.float32),
            pltpu.VMEM((_LANES,), jnp.float32),
            pltpu.SemaphoreType.DMA,
            pltpu.SemaphoreType.DMA,
            pltpu.SemaphoreType.DMA,
            pltpu.SemaphoreType.DMA,
            pltpu.SemaphoreType.DMA,
            pltpu.SemaphoreType.DMA,
        ],
        compiler_params=pltpu.CompilerParams(needs_layout_passes=False),
    )
    def run(d_hbm, r_hbm, t_hbm, inv_hbm, out_hbm,
            rb0, rb1, tb0, tb1, acct, ob0, ob1, invv,
            isem0, isem1, dsem0, dsem1, osem0, osem1):
        wid = lax.axis_index("s") * 2 + lax.axis_index("c")
        pltpu.sync_copy(inv_hbm.at[wid], invv)
        ival = invv[...]
        rb = (rb0, rb1)
        tb = (tb0, tb1)
        ob = (ob0, ob1)
        isem = (isem0, isem1)
        dsem = (dsem0, dsem1)
        osem = (osem0, osem1)
        w_row0 = wid * rows_per_w
        iota = lax.iota(jnp.int32, _LANES)
        base_diag = iota * (N + 1)
        zero16 = jnp.zeros((_LANES,), jnp.float32)

        def fire_in(g, buf):
            row = w_row0 + g * G
            for l in range(G):
                pltpu.async_copy(r_hbm.at[row + l],
                                 rb[buf].at[pl.ds(l * N, N)], isem[buf])
                pltpu.async_copy(t_hbm.at[row + l],
                                 tb[buf].at[pl.ds(l * N, N)], isem[buf])

        def fire_desc(g, buf):
            row = w_row0 + g * G
            pltpu.async_copy(d_hbm.at[pl.ds(row, G)], ob[buf], dsem[buf])

        def wait_in(g, buf):
            row = w_row0 + g * G
            for l in range(G):
                pltpu.make_async_copy(r_hbm.at[row + l],
                                      rb[buf].at[pl.ds(l * N, N)], isem[buf]).wait()
                pltpu.make_async_copy(t_hbm.at[row + l],
                                      tb[buf].at[pl.ds(l * N, N)], isem[buf]).wait()
            pltpu.make_async_copy(d_hbm.at[pl.ds(row, G)], ob[buf], dsem[buf]).wait()

        def fire_out(g, buf):
            row = w_row0 + g * G
            pltpu.async_copy(ob[buf], out_hbm.at[pl.ds(row, G)], osem[buf])

        def wait_out(buf):
            pltpu.make_async_copy(d_hbm.at[pl.ds(0, G)], ob[buf], osem[buf]).wait()

        def scat(rbr, tbr, aflat, ms):
            del ms
            rv = plsc.load_gather(rbr, [aflat])
            tv = plsc.load_gather(tbr, [aflat])
            sidx = (rv << 8) + (tv << 4) + iota
            plsc.addupdate_scatter(acct, [sidx], ival)

        def compute(buf):
            rbr, tbr, obr = rb[buf], tb[buf], ob[buf]

            @plsc.parallel_loop(0, m_main, 8)
            def col_body(m):
                mvec = base_diag + m
                for u in range(8):
                    scat(rbr, tbr, mvec + u, m + u)

            @plsc.parallel_loop(m_main, N, 1)
            def tail_body(m):
                aflat = base_diag + m - jnp.where(iota + m >= N, N, 0)
                scat(rbr, tbr, aflat, m)

            @plsc.parallel_loop(0, NB, 1)
            def trans_body(s):
                t16 = s & (NB - G)
                d = s & (G - 1)
                bvec = ((iota + d) & (G - 1)) + t16
                v = plsc.load_gather(acct, [(bvec << 4) + iota])
                plsc.addupdate_scatter(obr, [iota, bvec], v)

            @plsc.parallel_loop(0, NB, 1)
            def zero_body(s):
                off = pl.multiple_of(s * G, G)
                acct[pl.ds(off, G)] = zero16

        @plsc.parallel_loop(0, NB, 1)
        def zero_init(s):
            off = pl.multiple_of(s * G, G)
            acct[pl.ds(off, G)] = zero16

        fire_in(0, 0)
        fire_desc(0, 0)

        def step(k, carry):
            g0 = 2 * k
            g1 = g0 + 1
            fire_in(g1, 1)
            wait_in(g0, 0)
            compute(0)

            @pl.when(k >= 1)
            def _():
                wait_out(1)
            fire_desc(g1, 1)
            fire_out(g0, 0)

            @pl.when(k < n_iters - 1)
            def _():
                fire_in(g0 + 2, 0)
            wait_in(g1, 1)
            compute(1)

            @pl.when(k < n_iters - 1)
            def _():
                wait_out(0)
                fire_desc(g0 + 2, 0)
            fire_out(g1, 1)
            return carry

        lax.fori_loop(0, n_iters, step, 0)
        wait_out(0)
        wait_out(1)

    return run(d2, r2, t2, inv_splat).reshape(B, N, NB)
